# baseline (speedup 1.0000x reference)
import functools

import jax
import jax.numpy as jnp
from jax import lax
from jax.experimental import pallas as pl
from jax.experimental.pallas import tpu as pltpu
from jax.experimental.pallas import tpu_sc as plsc

_NC = 2
_NS = 16
_NW = _NC * _NS
_K = 100
_KF = 112
_G = 128


def _sc_mesh():
  return plsc.VectorSubcoreMesh(core_axis_name="c", subcore_axis_name="s",
                                num_cores=_NC, num_subcores=_NS)


def _deg_partials(col4, N):
  nseg, seg = col4.shape[1], col4.shape[2]
  npad = ((N + 2047) // 2048) * 2048
  span = npad // _NS

  @functools.partial(
      pl.kernel,
      mesh=_sc_mesh(),
      out_type=jax.ShapeDtypeStruct((_NC, 1, npad), jnp.float32),
      scratch_types=[
          pltpu.VMEM((span,), jnp.float32),
          pltpu.VMEM((seg, _K), jnp.int32),
          pltpu.VMEM((_KF,), jnp.float32),
          pltpu.VMEM_SHARED((npad,), jnp.float32),
      ],
  )
  def deg_kernel(col_hbm, out_hbm, zbuf, colv, ones, acc):
    cid = lax.axis_index("c")
    sid = lax.axis_index("s")
    wid = cid * _NS + sid
    for k in range(span // 16):
      zbuf[pl.ds(k * 16, 16)] = jnp.zeros((16,), jnp.float32)
    for k in range(_KF // 16):
      ones[pl.ds(k * 16, 16)] = jnp.ones((16,), jnp.float32)
    off = pl.multiple_of(sid * span, 128)
    pltpu.sync_copy(zbuf, acc.at[pl.ds(off, span)])
    plsc.subcore_barrier()

    for s in range(nseg):
      pltpu.sync_copy(col_hbm.at[wid, s], colv)

      def body(j, carry):
        pltpu.sync_copy(ones.at[pl.ds(0, _K)], acc.at[colv.at[j]], add=True)
        return carry

      lax.fori_loop(0, seg, body, 0)

    plsc.subcore_barrier()
    pltpu.sync_copy(acc.at[pl.ds(off, span)],
                    out_hbm.at[cid, 0, pl.ds(off, span)])

  return deg_kernel(col4)


def _mp_partials(y, row4, col4):
  N, H = y.shape
  nseg, seg = row4.shape[1], row4.shape[2]
  span = ((N // _NS + 7) // 8) * 8
  last = N - span * (_NS - 1)

  @functools.partial(
      pl.kernel,
      mesh=_sc_mesh(),
      out_type=jax.ShapeDtypeStruct((_NC, N, H), jnp.float32),
      scratch_types=[
          pltpu.VMEM((2, seg, _K), jnp.int32),
          pltpu.VMEM((2, seg, _K), jnp.int32),
          pltpu.VMEM((2, _K, H), jnp.float32),
          pltpu.VMEM_SHARED((N, H), jnp.float32),
          pltpu.SemaphoreType.DMA,
          pltpu.SemaphoreType.DMA,
          pltpu.SemaphoreType.DMA,
          pltpu.SemaphoreType.DMA,
          pltpu.SemaphoreType.DMA,
      ],
  )
  def mp_kernel(y_hbm, row_hbm, col_hbm, out_hbm, rowv, colv, rbuf, acc,
                gsem0, gsem1, ssem0, ssem1, isem):
    cid = lax.axis_index("c")
    sid = lax.axis_index("s")
    wid = cid * _NS + sid
    off = pl.multiple_of(sid * span, 8)

    @pl.when(sid < _NS - 1)
    def _init_main():
      pltpu.sync_copy(y_hbm.at[pl.ds(off, span)], acc.at[pl.ds(off, span)])

    @pl.when(sid == _NS - 1)
    def _init_last():
      pltpu.sync_copy(y_hbm.at[pl.ds(span * (_NS - 1), last)],
                      acc.at[pl.ds(span * (_NS - 1), last)])

    def _load_idx(s, ib):
      pltpu.async_copy(row_hbm.at[wid, s], rowv.at[ib], isem)
      pltpu.async_copy(col_hbm.at[wid, s], colv.at[ib], isem)

    def _wait_idx(s, ib):
      pltpu.make_async_copy(row_hbm.at[wid, s], rowv.at[ib], isem).wait()
      pltpu.make_async_copy(col_hbm.at[wid, s], colv.at[ib], isem).wait()

    _load_idx(0, 0)
    _wait_idx(0, 0)
    plsc.subcore_barrier()

    gsems = (gsem0, gsem1)
    ssems = (ssem0, ssem1)

    def _gather(ib, j, b):
      pltpu.async_copy(y_hbm.at[rowv.at[ib, j]], rbuf.at[b], gsems[b])

    def _wait_g(ib, j, b):
      pltpu.make_async_copy(y_hbm.at[rowv.at[ib, j]], rbuf.at[b],
                            gsems[b]).wait()

    def _scatter(ib, j, b):
      pltpu.async_copy(rbuf.at[b], acc.at[colv.at[ib, j]], ssems[b],
                       add=True)

    def _wait_s(ib, j, b):
      pltpu.make_async_copy(rbuf.at[b], acc.at[colv.at[ib, j]],
                            ssems[b]).wait()

    for s in range(nseg):
      ib = s % 2
      if s + 1 < nseg:
        _load_idx(s + 1, 1 - ib)
      _gather(ib, 0, 0)
      _gather(ib, 1, 1)

      def body(p, carry, ib=ib):
        j = p * 2
        _wait_g(ib, j, 0)
        _scatter(ib, j, 0)
        _wait_g(ib, j + 1, 1)
        _scatter(ib, j + 1, 1)
        _wait_s(ib, j, 0)
        _gather(ib, j + 2, 0)
        _wait_s(ib, j + 1, 1)
        _gather(ib, j + 3, 1)
        return carry

      lax.fori_loop(0, seg // 2 - 1, body, 0)
      _wait_g(ib, seg - 2, 0)
      _scatter(ib, seg - 2, 0)
      _wait_g(ib, seg - 1, 1)
      _scatter(ib, seg - 1, 1)
      _wait_s(ib, seg - 2, 0)
      _wait_s(ib, seg - 1, 1)
      if s + 1 < nseg:
        _wait_idx(s + 1, 1 - ib)

    plsc.subcore_barrier()

    @pl.when(sid < _NS - 1)
    def _out_main():
      pltpu.sync_copy(acc.at[pl.ds(off, span)],
                      out_hbm.at[cid, pl.ds(off, span)])

    @pl.when(sid == _NS - 1)
    def _out_last():
      pltpu.sync_copy(acc.at[pl.ds(span * (_NS - 1), last)],
                      out_hbm.at[cid, pl.ds(span * (_NS - 1), last)])

  return mp_kernel(y, row4, col4)


_BN = 2000


def _tc_in(x, W, dinv):
  N, D = x.shape
  H = W.shape[1]
  BN = _BN

  def body(xr, wr, dr, orf):
    orf[...] = jnp.dot(xr[...], wr[...],
                       preferred_element_type=jnp.float32) * dr[...]

  return pl.pallas_call(
      body,
      grid=(N // BN,),
      in_specs=[
          pl.BlockSpec((BN, D), lambda i: (i, 0)),
          pl.BlockSpec((D, H), lambda i: (0, 0)),
          pl.BlockSpec((BN, 1), lambda i: (i, 0)),
      ],
      out_specs=pl.BlockSpec((BN, H), lambda i: (i, 0)),
      out_shape=jax.ShapeDtypeStruct((N, H), jnp.float32),
  )(x, W, dinv)


def _tc_mid(acc, y, dinv, b, W):
  N, H = y.shape
  H2 = W.shape[1]
  BN = _BN

  def body(ar, yr, dr, br, wr, orf):
    h = jnp.maximum((ar[0] + ar[1] - yr[...]) * dr[...] + br[...], 0.0)
    orf[...] = jnp.dot(h, wr[...],
                       preferred_element_type=jnp.float32) * dr[...]

  return pl.pallas_call(
      body,
      grid=(N // BN,),
      in_specs=[
          pl.BlockSpec((2, BN, H), lambda i: (0, i, 0)),
          pl.BlockSpec((BN, H), lambda i: (i, 0)),
          pl.BlockSpec((BN, 1), lambda i: (i, 0)),
          pl.BlockSpec((1, H), lambda i: (0, 0)),
          pl.BlockSpec((H, H2), lambda i: (0, 0)),
      ],
      out_specs=pl.BlockSpec((BN, H2), lambda i: (i, 0)),
      out_shape=jax.ShapeDtypeStruct((N, H2), jnp.float32),
  )(acc, y, dinv, b, W)


def _tc_pool(acc, y, dinv, b, batch3, Wc, bc):
  N, H = y.shape
  O = Wc.shape[1]
  BN = _BN

  def body(ar, yr, dr, br, batchr, wcr, bcr, orf, sums, counts):
    i = pl.program_id(0)

    @pl.when(i == 0)
    def _init():
      sums[...] = jnp.zeros_like(sums)
      counts[...] = jnp.zeros_like(counts)

    h = jnp.maximum((ar[0] + ar[1] - yr[...]) * dr[...] + br[...], 0.0)
    seg = batchr[0]
    gids = lax.broadcasted_iota(jnp.int32, (_G, 1), 0)
    m = jnp.where(seg == gids, 1.0, 0.0)
    sums[...] += jnp.dot(m, h, preferred_element_type=jnp.float32)
    counts[...] += jnp.sum(m, axis=1, keepdims=True)

    @pl.when(i == pl.num_programs(0) - 1)
    def _fin():
      hg = sums[...] / jnp.maximum(counts[...], 1.0)
      orf[...] = jnp.dot(hg, wcr[...],
                         preferred_element_type=jnp.float32) + bcr[...]

  return pl.pallas_call(
      body,
      grid=(N // BN,),
      in_specs=[
          pl.BlockSpec((2, BN, H), lambda i: (0, i, 0)),
          pl.BlockSpec((BN, H), lambda i: (i, 0)),
          pl.BlockSpec((BN, 1), lambda i: (i, 0)),
          pl.BlockSpec((1, H), lambda i: (0, 0)),
          pl.BlockSpec((1, 1, BN), lambda i: (i, 0, 0)),
          pl.BlockSpec((H, O), lambda i: (0, 0)),
          pl.BlockSpec((1, O), lambda i: (0, 0)),
      ],
      out_specs=pl.BlockSpec((_G, O), lambda i: (0, 0)),
      out_shape=jax.ShapeDtypeStruct((_G, O), jnp.float32),
      scratch_shapes=[
          pltpu.VMEM((_G, H), jnp.float32),
          pltpu.VMEM((_G, 1), jnp.float32),
      ],
  )(acc, y, dinv, b, batch3, Wc, bc)


def kernel(x, edge_index, batch, W1, b1, W2, b2, Wc, bc):
  N, D = x.shape
  E = edge_index.shape[1]
  H = W1.shape[1]
  assert E % (_NW * _K) == 0 and N % _NS == 0 and N % _BN == 0

  seg = 20
  nseg = E // (_NW * _K * seg)
  row4 = edge_index[0].reshape(_NW, nseg, seg, _K)
  col4 = edge_index[1].reshape(_NW, nseg, seg, _K)

  degp = _deg_partials(col4, N)
  dinv = lax.rsqrt(degp[0, 0, :N] + degp[1, 0, :N] + 1.0)[:, None]

  y1 = _tc_in(x, W1, dinv)
  a1 = _mp_partials(y1, row4, col4)
  y2 = _tc_mid(a1, y1, dinv, b1.reshape(1, H), W2)
  a2 = _mp_partials(y2, row4, col4)
  batch3 = batch.reshape(N // _BN, 1, _BN)
  return _tc_pool(a2, y2, dinv, b2.reshape(1, H), batch3, Wc,
                  bc.reshape(1, -1))


# speedup vs baseline: 1.2084x; 1.2084x over previous
import functools

import jax
import jax.numpy as jnp
from jax import lax
from jax.experimental import pallas as pl
from jax.experimental.pallas import tpu as pltpu
from jax.experimental.pallas import tpu_sc as plsc

_NC = 2
_NS = 16
_NW = _NC * _NS
_K = 100
_KF = 112
_G = 128


def _sc_mesh():
  return plsc.VectorSubcoreMesh(core_axis_name="c", subcore_axis_name="s",
                                num_cores=_NC, num_subcores=_NS)


def _deg_partials(col4, N):
  nseg, seg = col4.shape[1], col4.shape[2]
  npad = ((N + 2047) // 2048) * 2048
  span = npad // _NS

  @functools.partial(
      pl.kernel,
      mesh=_sc_mesh(),
      out_type=jax.ShapeDtypeStruct((_NC, 1, npad), jnp.float32),
      scratch_types=[
          pltpu.VMEM((span,), jnp.float32),
          pltpu.VMEM((seg, _K), jnp.int32),
          pltpu.VMEM((_KF,), jnp.float32),
          pltpu.VMEM_SHARED((npad,), jnp.float32),
      ],
  )
  def deg_kernel(col_hbm, out_hbm, zbuf, colv, ones, acc):
    cid = lax.axis_index("c")
    sid = lax.axis_index("s")
    wid = cid * _NS + sid
    for k in range(span // 16):
      zbuf[pl.ds(k * 16, 16)] = jnp.zeros((16,), jnp.float32)
    for k in range(_KF // 16):
      ones[pl.ds(k * 16, 16)] = jnp.ones((16,), jnp.float32)
    off = pl.multiple_of(sid * span, 128)
    pltpu.sync_copy(zbuf, acc.at[pl.ds(off, span)])
    plsc.subcore_barrier()

    for s in range(nseg):
      pltpu.sync_copy(col_hbm.at[wid, s], colv)

      def body(j, carry):
        pltpu.sync_copy(ones.at[pl.ds(0, _K)], acc.at[colv.at[j]], add=True)
        return carry

      lax.fori_loop(0, seg, body, 0)

    plsc.subcore_barrier()
    pltpu.sync_copy(acc.at[pl.ds(off, span)],
                    out_hbm.at[cid, 0, pl.ds(off, span)])

  return deg_kernel(col4)


def _mp_partials(y, row4, col4):
  N, H = y.shape
  nseg, seg = row4.shape[1], row4.shape[2]
  span = ((N // _NS + 7) // 8) * 8
  last = N - span * (_NS - 1)

  @functools.partial(
      pl.kernel,
      mesh=_sc_mesh(),
      out_type=jax.ShapeDtypeStruct((_NC, N, H), jnp.float32),
      scratch_types=[
          pltpu.VMEM((2, seg, _K), jnp.int32),
          pltpu.VMEM((2, seg, _K), jnp.int32),
          pltpu.VMEM((2, _K, H), jnp.float32),
          pltpu.VMEM_SHARED((N, H), jnp.float32),
          pltpu.SemaphoreType.DMA,
          pltpu.SemaphoreType.DMA,
          pltpu.SemaphoreType.DMA,
          pltpu.SemaphoreType.DMA,
          pltpu.SemaphoreType.DMA,
      ],
  )
  def mp_kernel(y_hbm, row_hbm, col_hbm, out_hbm, rowv, colv, rbuf, acc,
                gsem0, gsem1, ssem0, ssem1, isem):
    cid = lax.axis_index("c")
    sid = lax.axis_index("s")
    wid = cid * _NS + sid
    off = pl.multiple_of(sid * span, 8)

    @pl.when(sid < _NS - 1)
    def _init_main():
      pltpu.sync_copy(y_hbm.at[pl.ds(off, span)], acc.at[pl.ds(off, span)])

    @pl.when(sid == _NS - 1)
    def _init_last():
      pltpu.sync_copy(y_hbm.at[pl.ds(span * (_NS - 1), last)],
                      acc.at[pl.ds(span * (_NS - 1), last)])

    def _load_idx(s, ib):
      pltpu.async_copy(row_hbm.at[wid, s], rowv.at[ib], isem)
      pltpu.async_copy(col_hbm.at[wid, s], colv.at[ib], isem)

    def _wait_idx(s, ib):
      pltpu.make_async_copy(row_hbm.at[wid, s], rowv.at[ib], isem).wait()
      pltpu.make_async_copy(col_hbm.at[wid, s], colv.at[ib], isem).wait()

    _load_idx(0, 0)
    _wait_idx(0, 0)
    plsc.subcore_barrier()

    gsems = (gsem0, gsem1)
    ssems = (ssem0, ssem1)

    def _gather(ib, j, b):
      pltpu.async_copy(y_hbm.at[rowv.at[ib, j]], rbuf.at[b], gsems[b])

    def _wait_g(ib, j, b):
      pltpu.make_async_copy(y_hbm.at[rowv.at[ib, j]], rbuf.at[b],
                            gsems[b]).wait()

    def _scatter(ib, j, b):
      pltpu.async_copy(rbuf.at[b], acc.at[colv.at[ib, j]], ssems[b],
                       add=True)

    def _wait_s(ib, j, b):
      pltpu.make_async_copy(rbuf.at[b], acc.at[colv.at[ib, j]],
                            ssems[b]).wait()

    def _scatter_sync(ib, j, b):
      pltpu.sync_copy(rbuf.at[b], acc.at[colv.at[ib, j]], add=True)

    for s in range(nseg):
      ib = s % 2
      if s + 1 < nseg:
        _load_idx(s + 1, 1 - ib)
      _gather(ib, 0, 0)

      def body(p, carry, ib=ib):
        j = p * 2
        _gather(ib, j + 1, 1)
        _wait_g(ib, j, 0)
        _scatter_sync(ib, j, 0)
        _gather(ib, j + 2, 0)
        _wait_g(ib, j + 1, 1)
        _scatter_sync(ib, j + 1, 1)
        return carry

      lax.fori_loop(0, (seg - 2) // 2, body, 0)
      _gather(ib, seg - 1, 1)
      _wait_g(ib, seg - 2, 0)
      _scatter_sync(ib, seg - 2, 0)
      _wait_g(ib, seg - 1, 1)
      _scatter_sync(ib, seg - 1, 1)
      if s + 1 < nseg:
        _wait_idx(s + 1, 1 - ib)

    plsc.subcore_barrier()

    @pl.when(sid < _NS - 1)
    def _out_main():
      pltpu.sync_copy(acc.at[pl.ds(off, span)],
                      out_hbm.at[cid, pl.ds(off, span)])

    @pl.when(sid == _NS - 1)
    def _out_last():
      pltpu.sync_copy(acc.at[pl.ds(span * (_NS - 1), last)],
                      out_hbm.at[cid, pl.ds(span * (_NS - 1), last)])

  return mp_kernel(y, row4, col4)


_BN = 2000


def _tc_in(x, W, dinv):
  N, D = x.shape
  H = W.shape[1]
  BN = _BN

  def body(xr, wr, dr, orf):
    orf[...] = jnp.dot(xr[...], wr[...],
                       preferred_element_type=jnp.float32) * dr[...]

  return pl.pallas_call(
      body,
      grid=(N // BN,),
      in_specs=[
          pl.BlockSpec((BN, D), lambda i: (i, 0)),
          pl.BlockSpec((D, H), lambda i: (0, 0)),
          pl.BlockSpec((BN, 1), lambda i: (i, 0)),
      ],
      out_specs=pl.BlockSpec((BN, H), lambda i: (i, 0)),
      out_shape=jax.ShapeDtypeStruct((N, H), jnp.float32),
  )(x, W, dinv)


def _tc_mid(acc, y, dinv, b, W):
  N, H = y.shape
  H2 = W.shape[1]
  BN = _BN

  def body(ar, yr, dr, br, wr, orf):
    h = jnp.maximum((ar[0] + ar[1] - yr[...]) * dr[...] + br[...], 0.0)
    orf[...] = jnp.dot(h, wr[...],
                       preferred_element_type=jnp.float32) * dr[...]

  return pl.pallas_call(
      body,
      grid=(N // BN,),
      in_specs=[
          pl.BlockSpec((2, BN, H), lambda i: (0, i, 0)),
          pl.BlockSpec((BN, H), lambda i: (i, 0)),
          pl.BlockSpec((BN, 1), lambda i: (i, 0)),
          pl.BlockSpec((1, H), lambda i: (0, 0)),
          pl.BlockSpec((H, H2), lambda i: (0, 0)),
      ],
      out_specs=pl.BlockSpec((BN, H2), lambda i: (i, 0)),
      out_shape=jax.ShapeDtypeStruct((N, H2), jnp.float32),
  )(acc, y, dinv, b, W)


def _tc_pool(acc, y, dinv, b, batch3, Wc, bc):
  N, H = y.shape
  O = Wc.shape[1]
  BN = _BN

  def body(ar, yr, dr, br, batchr, wcr, bcr, orf, sums, counts):
    i = pl.program_id(0)

    @pl.when(i == 0)
    def _init():
      sums[...] = jnp.zeros_like(sums)
      counts[...] = jnp.zeros_like(counts)

    h = jnp.maximum((ar[0] + ar[1] - yr[...]) * dr[...] + br[...], 0.0)
    seg = batchr[0]
    gids = lax.broadcasted_iota(jnp.int32, (_G, 1), 0)
    m = jnp.where(seg == gids, 1.0, 0.0)
    sums[...] += jnp.dot(m, h, preferred_element_type=jnp.float32)
    counts[...] += jnp.sum(m, axis=1, keepdims=True)

    @pl.when(i == pl.num_programs(0) - 1)
    def _fin():
      hg = sums[...] / jnp.maximum(counts[...], 1.0)
      orf[...] = jnp.dot(hg, wcr[...],
                         preferred_element_type=jnp.float32) + bcr[...]

  return pl.pallas_call(
      body,
      grid=(N // BN,),
      in_specs=[
          pl.BlockSpec((2, BN, H), lambda i: (0, i, 0)),
          pl.BlockSpec((BN, H), lambda i: (i, 0)),
          pl.BlockSpec((BN, 1), lambda i: (i, 0)),
          pl.BlockSpec((1, H), lambda i: (0, 0)),
          pl.BlockSpec((1, 1, BN), lambda i: (i, 0, 0)),
          pl.BlockSpec((H, O), lambda i: (0, 0)),
          pl.BlockSpec((1, O), lambda i: (0, 0)),
      ],
      out_specs=pl.BlockSpec((_G, O), lambda i: (0, 0)),
      out_shape=jax.ShapeDtypeStruct((_G, O), jnp.float32),
      scratch_shapes=[
          pltpu.VMEM((_G, H), jnp.float32),
          pltpu.VMEM((_G, 1), jnp.float32),
      ],
  )(acc, y, dinv, b, batch3, Wc, bc)


def kernel(x, edge_index, batch, W1, b1, W2, b2, Wc, bc):
  N, D = x.shape
  E = edge_index.shape[1]
  H = W1.shape[1]
  assert E % (_NW * _K) == 0 and N % _NS == 0 and N % _BN == 0

  seg = 20
  nseg = E // (_NW * _K * seg)
  row4 = edge_index[0].reshape(_NW, nseg, seg, _K)
  col4 = edge_index[1].reshape(_NW, nseg, seg, _K)

  degp = _deg_partials(col4, N)
  dinv = lax.rsqrt(degp[0, 0, :N] + degp[1, 0, :N] + 1.0)[:, None]

  y1 = _tc_in(x, W1, dinv)
  a1 = _mp_partials(y1, row4, col4)
  y2 = _tc_mid(a1, y1, dinv, b1.reshape(1, H), W2)
  a2 = _mp_partials(y2, row4, col4)
  batch3 = batch.reshape(N // _BN, 1, _BN)
  return _tc_pool(a2, y2, dinv, b2.reshape(1, H), batch3, Wc,
                  bc.reshape(1, -1))


# speedup vs baseline: 1.2349x; 1.0219x over previous
import functools

import jax
import jax.numpy as jnp
from jax import lax
from jax.experimental import pallas as pl
from jax.experimental.pallas import tpu as pltpu
from jax.experimental.pallas import tpu_sc as plsc

_NC = 2
_NS = 16
_NW = _NC * _NS
_K = 100
_KF = 112
_G = 128


def _sc_mesh():
  return plsc.VectorSubcoreMesh(core_axis_name="c", subcore_axis_name="s",
                                num_cores=_NC, num_subcores=_NS)


def _deg_partials(col3, N):
  cpw = col3.shape[1]
  npad = ((N + 2047) // 2048) * 2048
  span = npad // _NS

  @functools.partial(
      pl.kernel,
      mesh=_sc_mesh(),
      out_type=jax.ShapeDtypeStruct((_NC, 1, npad), jnp.float32),
      scratch_types=[
          pltpu.VMEM((span,), jnp.float32),
          pltpu.VMEM((cpw, 128), jnp.int32),
          pltpu.VMEM((128,), jnp.float32),
          pltpu.VMEM_SHARED((npad,), jnp.float32),
      ],
  )
  def deg_kernel(col_hbm, out_hbm, zbuf, colv, ones, acc):
    cid = lax.axis_index("c")
    sid = lax.axis_index("s")
    wid = cid * _NS + sid
    for k in range(span // 16):
      zbuf[pl.ds(k * 16, 16)] = jnp.zeros((16,), jnp.float32)
    for k in range(8):
      ones[pl.ds(k * 16, 16)] = jnp.ones((16,), jnp.float32)
    off = pl.multiple_of(sid * span, 128)
    pltpu.sync_copy(zbuf, acc.at[pl.ds(off, span)])
    pltpu.sync_copy(col_hbm.at[wid], colv)
    plsc.subcore_barrier()

    def body(j, carry):
      pltpu.sync_copy(ones, acc.at[colv.at[j]], add=True)
      return carry

    lax.fori_loop(0, cpw, body, 0)
    plsc.subcore_barrier()
    pltpu.sync_copy(acc.at[pl.ds(off, span)],
                    out_hbm.at[cid, 0, pl.ds(off, span)])

  return deg_kernel(col3)


def _mp_partials(y, row4, col4):
  N, H = y.shape
  nseg, seg = row4.shape[1], row4.shape[2]
  span = ((N // _NS + 7) // 8) * 8
  last = N - span * (_NS - 1)

  @functools.partial(
      pl.kernel,
      mesh=_sc_mesh(),
      out_type=jax.ShapeDtypeStruct((_NC, N, H), jnp.float32),
      scratch_types=[
          pltpu.VMEM((2, seg, _K), jnp.int32),
          pltpu.VMEM((2, seg, _K), jnp.int32),
          pltpu.VMEM((2, _K, H), jnp.float32),
          pltpu.VMEM_SHARED((N, H), jnp.float32),
          pltpu.SemaphoreType.DMA,
          pltpu.SemaphoreType.DMA,
          pltpu.SemaphoreType.DMA,
          pltpu.SemaphoreType.DMA,
          pltpu.SemaphoreType.DMA,
      ],
  )
  def mp_kernel(y_hbm, row_hbm, col_hbm, out_hbm, rowv, colv, rbuf, acc,
                gsem0, gsem1, ssem0, ssem1, isem):
    cid = lax.axis_index("c")
    sid = lax.axis_index("s")
    wid = cid * _NS + sid
    off = pl.multiple_of(sid * span, 8)

    @pl.when(sid < _NS - 1)
    def _init_main():
      pltpu.sync_copy(y_hbm.at[pl.ds(off, span)], acc.at[pl.ds(off, span)])

    @pl.when(sid == _NS - 1)
    def _init_last():
      pltpu.sync_copy(y_hbm.at[pl.ds(span * (_NS - 1), last)],
                      acc.at[pl.ds(span * (_NS - 1), last)])

    def _load_idx(s, ib):
      pltpu.async_copy(row_hbm.at[wid, s], rowv.at[ib], isem)
      pltpu.async_copy(col_hbm.at[wid, s], colv.at[ib], isem)

    def _wait_idx(s, ib):
      pltpu.make_async_copy(row_hbm.at[wid, s], rowv.at[ib], isem).wait()
      pltpu.make_async_copy(col_hbm.at[wid, s], colv.at[ib], isem).wait()

    _load_idx(0, 0)
    _wait_idx(0, 0)
    plsc.subcore_barrier()

    gsems = (gsem0, gsem1)
    ssems = (ssem0, ssem1)

    def _gather(ib, j, b):
      pltpu.async_copy(y_hbm.at[rowv.at[ib, j]], rbuf.at[b], gsems[b])

    def _wait_g(ib, j, b):
      pltpu.make_async_copy(y_hbm.at[rowv.at[ib, j]], rbuf.at[b],
                            gsems[b]).wait()

    def _scatter(ib, j, b):
      pltpu.async_copy(rbuf.at[b], acc.at[colv.at[ib, j]], ssems[b],
                       add=True)

    def _wait_s(ib, j, b):
      pltpu.make_async_copy(rbuf.at[b], acc.at[colv.at[ib, j]],
                            ssems[b]).wait()

    def _scatter_sync(ib, j, b):
      pltpu.sync_copy(rbuf.at[b], acc.at[colv.at[ib, j]], add=True)

    for s in range(nseg):
      ib = s % 2
      if s + 1 < nseg:
        _load_idx(s + 1, 1 - ib)
      _gather(ib, 0, 0)

      def body(p, carry, ib=ib):
        j = p * 2
        _gather(ib, j + 1, 1)
        _wait_g(ib, j, 0)
        _scatter_sync(ib, j, 0)
        _gather(ib, j + 2, 0)
        _wait_g(ib, j + 1, 1)
        _scatter_sync(ib, j + 1, 1)
        return carry

      lax.fori_loop(0, (seg - 2) // 2, body, 0)
      _gather(ib, seg - 1, 1)
      _wait_g(ib, seg - 2, 0)
      _scatter_sync(ib, seg - 2, 0)
      _wait_g(ib, seg - 1, 1)
      _scatter_sync(ib, seg - 1, 1)
      if s + 1 < nseg:
        _wait_idx(s + 1, 1 - ib)

    plsc.subcore_barrier()

    @pl.when(sid < _NS - 1)
    def _out_main():
      pltpu.sync_copy(acc.at[pl.ds(off, span)],
                      out_hbm.at[cid, pl.ds(off, span)])

    @pl.when(sid == _NS - 1)
    def _out_last():
      pltpu.sync_copy(acc.at[pl.ds(span * (_NS - 1), last)],
                      out_hbm.at[cid, pl.ds(span * (_NS - 1), last)])

  return mp_kernel(y, row4, col4)


_BN = 2000


def _tc_in(x, W, dinv):
  N, D = x.shape
  H = W.shape[1]
  BN = _BN

  def body(xr, wr, dr, orf):
    orf[...] = jnp.dot(xr[...], wr[...],
                       preferred_element_type=jnp.float32) * dr[...]

  return pl.pallas_call(
      body,
      grid=(N // BN,),
      in_specs=[
          pl.BlockSpec((BN, D), lambda i: (i, 0)),
          pl.BlockSpec((D, H), lambda i: (0, 0)),
          pl.BlockSpec((BN, 1), lambda i: (i, 0)),
      ],
      out_specs=pl.BlockSpec((BN, H), lambda i: (i, 0)),
      out_shape=jax.ShapeDtypeStruct((N, H), jnp.float32),
  )(x, W, dinv)


def _tc_mid(acc, y, dinv, b, W):
  N, H = y.shape
  H2 = W.shape[1]
  BN = _BN

  def body(ar, yr, dr, br, wr, orf):
    h = jnp.maximum((ar[0] + ar[1] - yr[...]) * dr[...] + br[...], 0.0)
    orf[...] = jnp.dot(h, wr[...],
                       preferred_element_type=jnp.float32) * dr[...]

  return pl.pallas_call(
      body,
      grid=(N // BN,),
      in_specs=[
          pl.BlockSpec((2, BN, H), lambda i: (0, i, 0)),
          pl.BlockSpec((BN, H), lambda i: (i, 0)),
          pl.BlockSpec((BN, 1), lambda i: (i, 0)),
          pl.BlockSpec((1, H), lambda i: (0, 0)),
          pl.BlockSpec((H, H2), lambda i: (0, 0)),
      ],
      out_specs=pl.BlockSpec((BN, H2), lambda i: (i, 0)),
      out_shape=jax.ShapeDtypeStruct((N, H2), jnp.float32),
  )(acc, y, dinv, b, W)


def _tc_pool(acc, y, dinv, b, batch3, Wc, bc):
  N, H = y.shape
  O = Wc.shape[1]
  BN = _BN

  def body(ar, yr, dr, br, batchr, wcr, bcr, orf, sums, counts):
    i = pl.program_id(0)

    @pl.when(i == 0)
    def _init():
      sums[...] = jnp.zeros_like(sums)
      counts[...] = jnp.zeros_like(counts)

    h = jnp.maximum((ar[0] + ar[1] - yr[...]) * dr[...] + br[...], 0.0)
    seg = batchr[0]
    gids = lax.broadcasted_iota(jnp.int32, (_G, 1), 0)
    m = jnp.where(seg == gids, 1.0, 0.0)
    sums[...] += jnp.dot(m, h, preferred_element_type=jnp.float32)
    counts[...] += jnp.sum(m, axis=1, keepdims=True)

    @pl.when(i == pl.num_programs(0) - 1)
    def _fin():
      hg = sums[...] / jnp.maximum(counts[...], 1.0)
      orf[...] = jnp.dot(hg, wcr[...],
                         preferred_element_type=jnp.float32) + bcr[...]

  return pl.pallas_call(
      body,
      grid=(N // BN,),
      in_specs=[
          pl.BlockSpec((2, BN, H), lambda i: (0, i, 0)),
          pl.BlockSpec((BN, H), lambda i: (i, 0)),
          pl.BlockSpec((BN, 1), lambda i: (i, 0)),
          pl.BlockSpec((1, H), lambda i: (0, 0)),
          pl.BlockSpec((1, 1, BN), lambda i: (i, 0, 0)),
          pl.BlockSpec((H, O), lambda i: (0, 0)),
          pl.BlockSpec((1, O), lambda i: (0, 0)),
      ],
      out_specs=pl.BlockSpec((_G, O), lambda i: (0, 0)),
      out_shape=jax.ShapeDtypeStruct((_G, O), jnp.float32),
      scratch_shapes=[
          pltpu.VMEM((_G, H), jnp.float32),
          pltpu.VMEM((_G, 1), jnp.float32),
      ],
  )(acc, y, dinv, b, batch3, Wc, bc)


def kernel(x, edge_index, batch, W1, b1, W2, b2, Wc, bc):
  N, D = x.shape
  E = edge_index.shape[1]
  H = W1.shape[1]
  assert E % (_NW * _K) == 0 and N % _NS == 0 and N % _BN == 0

  seg = 20
  nseg = E // (_NW * _K * seg)
  row4 = edge_index[0].reshape(_NW, nseg, seg, _K)
  col4 = edge_index[1].reshape(_NW, nseg, seg, _K)

  epad = -(-E // (_NW * 128)) * (_NW * 128)
  sent = N + (jnp.arange(epad - E, dtype=jnp.int32) % 128)
  colp = jnp.concatenate([edge_index[1], sent]).reshape(_NW, -1, 128)

  degp = _deg_partials(colp, N)
  dinv = lax.rsqrt(degp[0, 0, :N] + degp[1, 0, :N] + 1.0)[:, None]

  y1 = _tc_in(x, W1, dinv)
  a1 = _mp_partials(y1, row4, col4)
  y2 = _tc_mid(a1, y1, dinv, b1.reshape(1, H), W2)
  a2 = _mp_partials(y2, row4, col4)
  batch3 = batch.reshape(N // _BN, 1, _BN)
  return _tc_pool(a2, y2, dinv, b2.reshape(1, H), batch3, Wc,
                  bc.reshape(1, -1))


# speedup vs baseline: 1.3767x; 1.1148x over previous
import functools

import jax
import jax.numpy as jnp
from jax import lax
from jax.experimental import pallas as pl
from jax.experimental.pallas import tpu as pltpu
from jax.experimental.pallas import tpu_sc as plsc

_NC = 2
_NS = 16
_NW = _NC * _NS
_G = 128


def _sc_mesh():
  return plsc.VectorSubcoreMesh(core_axis_name="c", subcore_axis_name="s",
                                num_cores=_NC, num_subcores=_NS)


def _deg_partials(col3, N):
  cpw = col3.shape[1]
  npad = ((N + 2047) // 2048) * 2048
  span = npad // _NS

  @functools.partial(
      pl.kernel,
      mesh=_sc_mesh(),
      out_type=jax.ShapeDtypeStruct((_NC, 1, npad), jnp.float32),
      scratch_types=[
          pltpu.VMEM((span,), jnp.float32),
          pltpu.VMEM((cpw, 128), jnp.int32),
          pltpu.VMEM((128,), jnp.float32),
          pltpu.VMEM_SHARED((npad,), jnp.float32),
      ],
  )
  def deg_kernel(col_hbm, out_hbm, zbuf, colv, ones, acc):
    cid = lax.axis_index("c")
    sid = lax.axis_index("s")
    wid = cid * _NS + sid
    for k in range(span // 16):
      zbuf[pl.ds(k * 16, 16)] = jnp.zeros((16,), jnp.float32)
    for k in range(8):
      ones[pl.ds(k * 16, 16)] = jnp.ones((16,), jnp.float32)
    off = pl.multiple_of(sid * span, 128)
    pltpu.sync_copy(zbuf, acc.at[pl.ds(off, span)])
    pltpu.sync_copy(col_hbm.at[wid], colv)
    plsc.subcore_barrier()

    def body(j, carry):
      pltpu.sync_copy(ones, acc.at[colv.at[j]], add=True)
      return carry

    lax.fori_loop(0, cpw, body, 0)
    plsc.subcore_barrier()
    pltpu.sync_copy(acc.at[pl.ds(off, span)],
                    out_hbm.at[cid, 0, pl.ds(off, span)])

  return deg_kernel(col3)


def _mp_partials(y, rowp, colp, ndum):
  N, H = y.shape
  cpw = rowp.shape[1]
  seg = 8
  nseg = cpw // seg
  span = ((N // _NS + 7) // 8) * 8
  last = N - span * (_NS - 1)

  @functools.partial(
      pl.kernel,
      mesh=_sc_mesh(),
      out_type=jax.ShapeDtypeStruct((_NC, N, H), jnp.float32),
      scratch_types=[
          pltpu.VMEM((2, seg, 128), jnp.int32),
          pltpu.VMEM((2, seg, 128), jnp.int32),
          pltpu.VMEM((2, 128, H), jnp.float32),
          pltpu.VMEM_SHARED((N + ndum, H), jnp.float32),
          pltpu.SemaphoreType.DMA,
          pltpu.SemaphoreType.DMA,
          pltpu.SemaphoreType.DMA,
      ],
  )
  def mp_kernel(y_hbm, row_hbm, col_hbm, out_hbm, rowv, colv, rbuf, acc,
                gsem0, gsem1, isem):
    cid = lax.axis_index("c")
    sid = lax.axis_index("s")
    wid = cid * _NS + sid
    off = pl.multiple_of(sid * span, 8)

    @pl.when(sid < _NS - 1)
    def _init_main():
      pltpu.sync_copy(y_hbm.at[pl.ds(off, span)], acc.at[pl.ds(off, span)])

    @pl.when(sid == _NS - 1)
    def _init_last():
      pltpu.sync_copy(y_hbm.at[pl.ds(span * (_NS - 1), last)],
                      acc.at[pl.ds(span * (_NS - 1), last)])

    def _load_idx(s, ib):
      sl = pl.ds(s * seg, seg)
      pltpu.async_copy(row_hbm.at[wid, sl], rowv.at[ib], isem)
      pltpu.async_copy(col_hbm.at[wid, sl], colv.at[ib], isem)

    def _wait_idx(s, ib):
      sl = pl.ds(s * seg, seg)
      pltpu.make_async_copy(row_hbm.at[wid, sl], rowv.at[ib], isem).wait()
      pltpu.make_async_copy(col_hbm.at[wid, sl], colv.at[ib], isem).wait()

    _load_idx(0, 0)
    _wait_idx(0, 0)
    plsc.subcore_barrier()

    gsems = (gsem0, gsem1)

    def _gather(ib, j, b):
      pltpu.async_copy(y_hbm.at[rowv.at[ib, j]], rbuf.at[b], gsems[b])

    def _wait_g(ib, j, b):
      pltpu.make_async_copy(y_hbm.at[rowv.at[ib, j]], rbuf.at[b],
                            gsems[b]).wait()

    def _scatter(ib, j, b):
      pltpu.sync_copy(rbuf.at[b], acc.at[colv.at[ib, j]], add=True)

    _gather(0, 0, 0)
    _gather(0, 1, 1)

    for s in range(nseg):
      ib = s % 2
      if s + 1 < nseg:
        _load_idx(s + 1, 1 - ib)

      def body(p, carry, ib=ib):
        j = p * 2
        _wait_g(ib, j, 0)
        _scatter(ib, j, 0)
        _gather(ib, j + 2, 0)
        _wait_g(ib, j + 1, 1)
        _scatter(ib, j + 1, 1)
        _gather(ib, j + 3, 1)
        return carry

      lax.fori_loop(0, (seg - 2) // 2, body, 0)
      if s + 1 < nseg:
        _wait_idx(s + 1, 1 - ib)
        _wait_g(ib, seg - 2, 0)
        _scatter(ib, seg - 2, 0)
        _gather(1 - ib, 0, 0)
        _wait_g(ib, seg - 1, 1)
        _scatter(ib, seg - 1, 1)
        _gather(1 - ib, 1, 1)
      else:
        _wait_g(ib, seg - 2, 0)
        _scatter(ib, seg - 2, 0)
        _wait_g(ib, seg - 1, 1)
        _scatter(ib, seg - 1, 1)

    plsc.subcore_barrier()

    @pl.when(sid < _NS - 1)
    def _out_main():
      pltpu.sync_copy(acc.at[pl.ds(off, span)],
                      out_hbm.at[cid, pl.ds(off, span)])

    @pl.when(sid == _NS - 1)
    def _out_last():
      pltpu.sync_copy(acc.at[pl.ds(span * (_NS - 1), last)],
                      out_hbm.at[cid, pl.ds(span * (_NS - 1), last)])

  return mp_kernel(y, rowp, colp)


_BN = 2000


def _tc_in(x, W, dinv):
  N, D = x.shape
  H = W.shape[1]
  BN = _BN

  def body(xr, wr, dr, orf):
    orf[...] = jnp.dot(xr[...], wr[...],
                       preferred_element_type=jnp.float32) * dr[...]

  return pl.pallas_call(
      body,
      grid=(N // BN,),
      in_specs=[
          pl.BlockSpec((BN, D), lambda i: (i, 0)),
          pl.BlockSpec((D, H), lambda i: (0, 0)),
          pl.BlockSpec((BN, 1), lambda i: (i, 0)),
      ],
      out_specs=pl.BlockSpec((BN, H), lambda i: (i, 0)),
      out_shape=jax.ShapeDtypeStruct((N, H), jnp.float32),
  )(x, W, dinv)


def _tc_mid(acc, y, dinv, b, W):
  N, H = y.shape
  H2 = W.shape[1]
  BN = _BN

  def body(ar, yr, dr, br, wr, orf):
    h = jnp.maximum((ar[0] + ar[1] - yr[...]) * dr[...] + br[...], 0.0)
    orf[...] = jnp.dot(h, wr[...],
                       preferred_element_type=jnp.float32) * dr[...]

  return pl.pallas_call(
      body,
      grid=(N // BN,),
      in_specs=[
          pl.BlockSpec((2, BN, H), lambda i: (0, i, 0)),
          pl.BlockSpec((BN, H), lambda i: (i, 0)),
          pl.BlockSpec((BN, 1), lambda i: (i, 0)),
          pl.BlockSpec((1, H), lambda i: (0, 0)),
          pl.BlockSpec((H, H2), lambda i: (0, 0)),
      ],
      out_specs=pl.BlockSpec((BN, H2), lambda i: (i, 0)),
      out_shape=jax.ShapeDtypeStruct((N, H2), jnp.float32),
  )(acc, y, dinv, b, W)


def _tc_pool(acc, y, dinv, b, batch3, Wc, bc):
  N, H = y.shape
  O = Wc.shape[1]
  BN = _BN

  def body(ar, yr, dr, br, batchr, wcr, bcr, orf, sums, counts):
    i = pl.program_id(0)

    @pl.when(i == 0)
    def _init():
      sums[...] = jnp.zeros_like(sums)
      counts[...] = jnp.zeros_like(counts)

    h = jnp.maximum((ar[0] + ar[1] - yr[...]) * dr[...] + br[...], 0.0)
    seg = batchr[0]
    gids = lax.broadcasted_iota(jnp.int32, (_G, 1), 0)
    m = jnp.where(seg == gids, 1.0, 0.0)
    sums[...] += jnp.dot(m, h, preferred_element_type=jnp.float32)
    counts[...] += jnp.sum(m, axis=1, keepdims=True)

    @pl.when(i == pl.num_programs(0) - 1)
    def _fin():
      hg = sums[...] / jnp.maximum(counts[...], 1.0)
      orf[...] = jnp.dot(hg, wcr[...],
                         preferred_element_type=jnp.float32) + bcr[...]

  return pl.pallas_call(
      body,
      grid=(N // BN,),
      in_specs=[
          pl.BlockSpec((2, BN, H), lambda i: (0, i, 0)),
          pl.BlockSpec((BN, H), lambda i: (i, 0)),
          pl.BlockSpec((BN, 1), lambda i: (i, 0)),
          pl.BlockSpec((1, H), lambda i: (0, 0)),
          pl.BlockSpec((1, 1, BN), lambda i: (i, 0, 0)),
          pl.BlockSpec((H, O), lambda i: (0, 0)),
          pl.BlockSpec((1, O), lambda i: (0, 0)),
      ],
      out_specs=pl.BlockSpec((_G, O), lambda i: (0, 0)),
      out_shape=jax.ShapeDtypeStruct((_G, O), jnp.float32),
      scratch_shapes=[
          pltpu.VMEM((_G, H), jnp.float32),
          pltpu.VMEM((_G, 1), jnp.float32),
      ],
  )(acc, y, dinv, b, batch3, Wc, bc)


def kernel(x, edge_index, batch, W1, b1, W2, b2, Wc, bc):
  N, D = x.shape
  E = edge_index.shape[1]
  H = W1.shape[1]
  assert E % _NW == 0 and N % _NS == 0 and N % _BN == 0

  ew = E // _NW
  ewp = -(-ew // 128) * 128
  npw = ewp - ew
  ndum = 64
  sentr = (jnp.arange(_NW * npw, dtype=jnp.int32) % N).reshape(_NW, npw)
  sentc = N + (jnp.arange(_NW * npw, dtype=jnp.int32) % ndum).reshape(
      _NW, npw)
  rowp = jnp.concatenate(
      [edge_index[0].reshape(_NW, ew), sentr], axis=1).reshape(_NW, -1, 128)
  colp = jnp.concatenate(
      [edge_index[1].reshape(_NW, ew), sentc], axis=1).reshape(_NW, -1, 128)

  degp = _deg_partials(colp, N)
  dinv = lax.rsqrt(degp[0, 0, :N] + degp[1, 0, :N] + 1.0)[:, None]

  y1 = _tc_in(x, W1, dinv)
  a1 = _mp_partials(y1, rowp, colp, ndum)
  y2 = _tc_mid(a1, y1, dinv, b1.reshape(1, H), W2)
  a2 = _mp_partials(y2, rowp, colp, ndum)
  batch3 = batch.reshape(N // _BN, 1, _BN)
  return _tc_pool(a2, y2, dinv, b2.reshape(1, H), batch3, Wc,
                  bc.reshape(1, -1))
